# baseline (speedup 1.0000x reference)
import functools

import jax
import jax.numpy as jnp
from jax import lax
from jax.experimental import pallas as pl
from jax.experimental.pallas import tpu as pltpu
from jax.experimental.pallas import tpu_sc as plsc

_TILE = 2048
_CHUNK = _TILE // 4
_NSEM = 8


def _sc_gather_rows(offset_embed, delay):
    n_rows, d = offset_embed.shape
    b = delay.shape[0]
    mesh = plsc.VectorSubcoreMesh(core_axis_name="c", subcore_axis_name="s")

    @functools.partial(
        pl.kernel,
        mesh=mesh,
        out_type=jax.ShapeDtypeStruct((b, d), jnp.float32),
        scratch_types=[
            pltpu.VMEM((b,), jnp.int32),
            pltpu.VMEM((b, d), jnp.float32),
            pltpu.SemaphoreType.DMA,
        ],
    )
    def gather(table_hbm, idx_hbm, out_hbm, idx_v, rows_v, sem):
        wid = lax.axis_index("s") * 2 + lax.axis_index("c")

        @pl.when(wid == 0)
        def _():
            pltpu.sync_copy(idx_hbm, idx_v)
            pltpu.async_copy(table_hbm.at[idx_v], rows_v, sem).wait()
            pltpu.sync_copy(rows_v, out_hbm)

    return gather(offset_embed, delay)


def _tc_body(x_ref, pe_ref, off_ref, out_ref, scratch, sems):
    t = pl.program_id(0)
    b = pl.program_id(1)
    n_b = pl.num_programs(1)
    step = t * n_b + b
    for h in range(4):
        chunk = 4 * step + h
        slot = jax.lax.rem(chunk, _NSEM)
        row0 = t * _TILE + h * _CHUNK

        @pl.when(chunk >= _NSEM)
        def _wait_prev():
            pltpu.make_async_copy(
                scratch.at[slot], out_ref.at[b, pl.ds(row0, _CHUNK), :], sems.at[slot]
            ).wait()

        scratch[slot] = (
            x_ref[0, h * _CHUNK : (h + 1) * _CHUNK]
            + pe_ref[h * _CHUNK : (h + 1) * _CHUNK]
            + off_ref[0]
        )
        pltpu.make_async_copy(
            scratch.at[slot], out_ref.at[b, pl.ds(row0, _CHUNK), :], sems.at[slot]
        ).start()

    @pl.when(step == pl.num_programs(0) * n_b - 1)
    def _drain():
        for k in range(_NSEM):
            pltpu.make_async_copy(
                scratch.at[k], out_ref.at[b, pl.ds(t * _TILE, _CHUNK), :], sems.at[k]
            ).wait()


def kernel(x, delay, offset_embed, sin_pe):
    B, T, D = x.shape
    pe = sin_pe[:T]
    off_rows = _sc_gather_rows(offset_embed, delay).reshape(B, 1, D)
    n_t = T // _TILE

    return pl.pallas_call(
        _tc_body,
        grid=(n_t, B),
        in_specs=[
            pl.BlockSpec((1, _TILE, D), lambda t, b: (b, t, 0)),
            pl.BlockSpec((_TILE, D), lambda t, b: (t, 0)),
            pl.BlockSpec((1, 1, D), lambda t, b: (b, 0, 0)),
        ],
        out_specs=pl.BlockSpec(memory_space=pltpu.MemorySpace.HBM),
        scratch_shapes=[
            pltpu.VMEM((_NSEM, _CHUNK, D), jnp.float32),
            pltpu.SemaphoreType.DMA((_NSEM,)),
        ],
        out_shape=jax.ShapeDtypeStruct((B, T, D), x.dtype),
    )(x, pe, off_rows)


# speedup vs baseline: 1.2723x; 1.2723x over previous
import jax
import jax.numpy as jnp
from jax.experimental import pallas as pl
from jax.experimental.pallas import tpu as pltpu

_MAX_DELAY = 8
_TILE = 2048
_CHUNK = _TILE // 4
_NSEM = 16


def _body(delay_ref, x_ref, pe_ref, off_ref, out_ref, scratch, sems):
    del delay_ref
    t = pl.program_id(0)
    b = pl.program_id(1)
    n_b = pl.num_programs(1)
    step = t * n_b + b
    for h in range(4):
        chunk = 4 * step + h
        slot = jax.lax.rem(chunk, _NSEM)
        row0 = t * _TILE + h * _CHUNK

        @pl.when(chunk >= _NSEM)
        def _wait_prev():
            pltpu.make_async_copy(
                scratch.at[slot], out_ref.at[b, pl.ds(row0, _CHUNK), :], sems.at[slot]
            ).wait()

        scratch[slot] = (
            x_ref[0, h * _CHUNK : (h + 1) * _CHUNK]
            + pe_ref[h * _CHUNK : (h + 1) * _CHUNK]
            + off_ref[0]
        )
        pltpu.make_async_copy(
            scratch.at[slot], out_ref.at[b, pl.ds(row0, _CHUNK), :], sems.at[slot]
        ).start()

    @pl.when(step == pl.num_programs(0) * n_b - 1)
    def _drain():
        for k in range(_NSEM):
            pltpu.make_async_copy(
                scratch.at[k], out_ref.at[b, pl.ds(t * _TILE, _CHUNK), :], sems.at[k]
            ).wait()


def kernel(x, delay, offset_embed, sin_pe):
    B, T, D = x.shape
    pe = sin_pe[:T]
    off3 = offset_embed.reshape(offset_embed.shape[0], 1, D)
    n_t = T // _TILE

    grid_spec = pltpu.PrefetchScalarGridSpec(
        num_scalar_prefetch=1,
        grid=(n_t, B),
        in_specs=[
            pl.BlockSpec((1, _TILE, D), lambda t, b, d: (b, t, 0)),
            pl.BlockSpec((_TILE, D), lambda t, b, d: (t, 0)),
            pl.BlockSpec((1, 1, D), lambda t, b, d: (jnp.clip(d[b], 0, _MAX_DELAY), 0, 0)),
        ],
        out_specs=pl.BlockSpec(memory_space=pltpu.MemorySpace.HBM),
        scratch_shapes=[
            pltpu.VMEM((_NSEM, _CHUNK, D), jnp.float32),
            pltpu.SemaphoreType.DMA((_NSEM,)),
        ],
    )
    return pl.pallas_call(
        _body,
        grid_spec=grid_spec,
        out_shape=jax.ShapeDtypeStruct((B, T, D), x.dtype),
    )(delay, x, pe, off3)


# speedup vs baseline: 1.2747x; 1.0019x over previous
import jax
import jax.numpy as jnp
from jax.experimental import pallas as pl
from jax.experimental.pallas import tpu as pltpu

_MAX_DELAY = 8
_TILE = 2048
_CHUNK = _TILE // 4
_NSEM = 8


def _body(delay_ref, x_ref, pe_ref, off_ref, out_ref, scratch, sems):
    del delay_ref
    t = pl.program_id(0)
    b = pl.program_id(1)
    n_b = pl.num_programs(1)
    step = t * n_b + b
    for h in range(4):
        chunk = 4 * step + h
        slot = jax.lax.rem(chunk, _NSEM)
        row0 = t * _TILE + h * _CHUNK

        @pl.when(chunk >= _NSEM)
        def _wait_prev():
            pltpu.make_async_copy(
                scratch.at[slot], out_ref.at[b, pl.ds(row0, _CHUNK), :], sems.at[slot]
            ).wait()

        scratch[slot] = (
            x_ref[0, h * _CHUNK : (h + 1) * _CHUNK]
            + pe_ref[h * _CHUNK : (h + 1) * _CHUNK]
            + off_ref[0]
        )
        pltpu.make_async_copy(
            scratch.at[slot], out_ref.at[b, pl.ds(row0, _CHUNK), :], sems.at[slot]
        ).start()

    @pl.when(step == pl.num_programs(0) * n_b - 1)
    def _drain():
        for k in range(_NSEM):
            pltpu.make_async_copy(
                scratch.at[k], out_ref.at[b, pl.ds(t * _TILE, _CHUNK), :], sems.at[k]
            ).wait()


def kernel(x, delay, offset_embed, sin_pe):
    B, T, D = x.shape
    pe = sin_pe[:T]
    off3 = offset_embed.reshape(offset_embed.shape[0], 1, D)
    n_t = T // _TILE

    grid_spec = pltpu.PrefetchScalarGridSpec(
        num_scalar_prefetch=1,
        grid=(n_t, B),
        in_specs=[
            pl.BlockSpec((1, _TILE, D), lambda t, b, d: (b, t, 0)),
            pl.BlockSpec((_TILE, D), lambda t, b, d: (t, 0)),
            pl.BlockSpec((1, 1, D), lambda t, b, d: (jnp.clip(d[b], 0, _MAX_DELAY), 0, 0)),
        ],
        out_specs=pl.BlockSpec(memory_space=pltpu.MemorySpace.HBM),
        scratch_shapes=[
            pltpu.VMEM((_NSEM, _CHUNK, D), jnp.float32),
            pltpu.SemaphoreType.DMA((_NSEM,)),
        ],
    )
    return pl.pallas_call(
        _body,
        grid_spec=grid_spec,
        out_shape=jax.ShapeDtypeStruct((B, T, D), x.dtype),
    )(delay, x, pe, off3)
